# baseline (speedup 1.0000x reference)
import jax
import jax.numpy as jnp
from jax.experimental import pallas as pl
from jax.experimental.pallas import tpu as pltpu

B, N, S, D, P = 4, 8, 8192, 64, 64
R = S // 2
RCHUNK = 1024
NJ = R // RCHUNK


def _router_body(hs_ref, w_ref, b_ref, out_ref, acc_ref):
    bi = pl.program_id(0)
    j = pl.program_id(1)

    @pl.when(jnp.logical_and(bi == 0, j == 0))
    def _init():
        acc_ref[...] = jnp.zeros_like(acc_ref)

    x = jnp.sum(hs_ref[0], axis=0)
    y = jax.lax.dot_general(
        x, w_ref[...], (((1,), (0,)), ((), ())),
        preferred_element_type=jnp.float32,
    )
    t = y + jnp.float32(N) * b_ref[...]
    cnt = jnp.sum((t > 0).astype(jnp.float32), axis=0)

    row = jax.lax.broadcasted_iota(jnp.int32, (8, 2 * P), 0)
    acc_ref[...] += jnp.where(row == bi, cnt[None, :], 0.0)

    @pl.when(jnp.logical_and(bi == B - 1, j == NJ - 1))
    def _finish():
        cnt2 = acc_ref[0:B, 0:P] + acc_ref[0:B, P:2 * P]
        g = cnt2 * jnp.float32(1.0 / S)
        z = jax.lax.dot_general(
            g, w_ref[0:D, 0:P], (((1,), (0,)), ((), ())),
            preferred_element_type=jnp.float32,
        ) + b_ref[0:1, 0:P]
        m = jnp.max(z, axis=1, keepdims=True)
        ii = jax.lax.broadcasted_iota(jnp.int32, (B, P), 1)
        idx = jnp.min(jnp.where(z == m, ii, P), axis=1, keepdims=True)
        out_ref[...] = (ii == idx).astype(jnp.int32)


def kernel(hidden_states, W, b):
    hs2 = hidden_states.reshape(B, N, R, 2 * D)
    wt = W.T
    zz = jnp.zeros((D, P), jnp.float32)
    wbig = jnp.block([[wt, zz], [zz, wt]])
    b2 = jnp.concatenate([b, b]).reshape(1, 2 * P)
    return pl.pallas_call(
        _router_body,
        grid=(B, NJ),
        in_specs=[
            pl.BlockSpec((1, N, RCHUNK, 2 * D), lambda bi, j: (bi, 0, j, 0)),
            pl.BlockSpec((2 * D, 2 * P), lambda bi, j: (0, 0)),
            pl.BlockSpec((1, 2 * P), lambda bi, j: (0, 0)),
        ],
        out_specs=pl.BlockSpec((B, P), lambda bi, j: (0, 0)),
        out_shape=jax.ShapeDtypeStruct((B, P), jnp.int32),
        scratch_shapes=[pltpu.VMEM((8, 2 * P), jnp.float32)],
    )(hs2, wbig, b2)


# speedup vs baseline: 1.4411x; 1.4411x over previous
import jax
import jax.numpy as jnp
from jax.experimental import pallas as pl
from jax.experimental.pallas import tpu as pltpu

B, N, S, D, P = 4, 8, 8192, 64, 64
SCHUNK = 2048
NJ = S // SCHUNK


def _router_body(hs_ref, w_ref, b_ref, out_ref, acc_ref):
    bi = pl.program_id(0)
    j = pl.program_id(1)

    @pl.when(jnp.logical_and(bi == 0, j == 0))
    def _init():
        acc_ref[...] = jnp.zeros_like(acc_ref)

    x = jnp.sum(hs_ref[0], axis=0)
    y = jax.lax.dot_general(
        x, w_ref[...], (((1,), (1,)), ((), ())),
        preferred_element_type=jnp.float32,
    )
    t = y + jnp.float32(N) * b_ref[...]
    cnt = jnp.sum((t > 0).astype(jnp.float32), axis=0)

    row = jax.lax.broadcasted_iota(jnp.int32, (8, P), 0)
    acc_ref[...] += jnp.where(row == bi, cnt[None, :], 0.0)

    @pl.when(jnp.logical_and(bi == B - 1, j == NJ - 1))
    def _finish():
        g = acc_ref[0:B, :] * jnp.float32(1.0 / S)
        z = jax.lax.dot_general(
            g, w_ref[...], (((1,), (1,)), ((), ())),
            preferred_element_type=jnp.float32,
        ) + b_ref[...]
        m = jnp.max(z, axis=1, keepdims=True)
        ii = jax.lax.broadcasted_iota(jnp.int32, (B, P), 1)
        idx = jnp.min(jnp.where(z == m, ii, P), axis=1, keepdims=True)
        out_ref[...] = (ii == idx).astype(jnp.int32)


def kernel(hidden_states, W, b):
    b2 = b.reshape(1, P)
    return pl.pallas_call(
        _router_body,
        grid=(B, NJ),
        in_specs=[
            pl.BlockSpec((1, N, SCHUNK, D), lambda bi, j: (bi, 0, j, 0)),
            pl.BlockSpec((P, D), lambda bi, j: (0, 0)),
            pl.BlockSpec((1, P), lambda bi, j: (0, 0)),
        ],
        out_specs=pl.BlockSpec((B, P), lambda bi, j: (0, 0)),
        out_shape=jax.ShapeDtypeStruct((B, P), jnp.int32),
        scratch_shapes=[pltpu.VMEM((8, P), jnp.float32)],
    )(hidden_states, W, b2)
